# baseline (speedup 1.0000x reference)
import functools

import jax
import jax.numpy as jnp
from jax import lax
from jax.experimental import pallas as pl
from jax.experimental.pallas import tpu as pltpu
from jax.experimental.pallas import tpu_sc as plsc

f32 = jnp.float32
i32 = jnp.int32

NCORES = 2
NSUB = 16
LANES = 16

EDGE_BLK = 400


def _tc_proj_kernel(x_ref, wext_ref, wad_ref, wasad_ref,
                    h1ext_ref, ad1_ref, asad_ref, mx_ref):
    i = pl.program_id(0)
    xb = x_ref[...]
    h1ext_ref[...] = jnp.dot(xb, wext_ref[...], preferred_element_type=f32)
    ad1_ref[...] = jnp.dot(xb, wad_ref[...], preferred_element_type=f32)
    asad = jnp.dot(xb, wasad_ref[...], preferred_element_type=f32)
    asad_ref[...] = asad
    m = jnp.max(asad, axis=0, keepdims=True)
    prev = jnp.where(i == 0, jnp.full_like(m, -jnp.inf), mx_ref[...])
    mx_ref[...] = jnp.maximum(prev, m)


def _tc_mid_kernel(nd_ref, h1ext_ref, asad_ref, sh_ref, b1_ref,
                   selsum_ref, sel64_ref, sel8_ref, rep_ref,
                   w2e_ref, c2_ref, w2ad_ref, w2asad_ref,
                   h2ext_ref, ad2_ref, mx_ref):
    i = pl.program_id(0)
    nd = nd_ref[...]
    tot = nd[0] + nd[1]
    h1e = h1ext_ref[...]
    asad = asad_ref[...]
    pre = jnp.dot(asad, selsum_ref[...], preferred_element_type=f32)
    pre = jnp.where(pre >= 0, pre, 0.2 * pre) - sh_ref[...]
    ws = jnp.exp(pre)
    h64 = jnp.dot(h1e, sel64_ref[...], preferred_element_type=f32)
    ws64 = jnp.dot(ws, rep_ref[...], preferred_element_type=f32)
    num = jnp.dot(tot, sel64_ref[...], preferred_element_type=f32) + ws64 * h64
    den = jnp.dot(tot, sel8_ref[...], preferred_element_type=f32) + ws
    den64 = jnp.dot(den, rep_ref[...], preferred_element_type=f32)
    out1 = num / (den64 + 1e-16) + b1_ref[...]
    hg = jnp.where(out1 > 0, out1, jnp.exp(jnp.minimum(out1, 0.0)) - 1.0)
    h2e = jnp.dot(hg, w2e_ref[...], preferred_element_type=f32) + c2_ref[...]
    h2ext_ref[...] = h2e
    ad2_ref[...] = jnp.dot(hg, w2ad_ref[...], preferred_element_type=f32)
    asad2 = jnp.dot(hg, w2asad_ref[...], preferred_element_type=f32)
    m = jnp.max(asad2, axis=0, keepdims=True)
    prev = jnp.where(i == 0, jnp.full_like(m, -jnp.inf), mx_ref[...])
    mx_ref[...] = jnp.maximum(prev, m)


def _tc_out_kernel(nd_ref, h2ext_ref, ad2_ref, sh_ref, b2_ref,
                   sel41_ref, e00_ref, u48_ref, seln_ref, seld_ref,
                   out_ref):
    nd = nd_ref[...]
    tot = nd[0] + nd[1]
    h2e = h2ext_ref[...]
    as2 = jnp.dot(h2e, sel41_ref[...], preferred_element_type=f32)
    ad2 = jnp.dot(ad2_ref[...], e00_ref[...], preferred_element_type=f32)
    pre = as2 + ad2
    pre = jnp.where(pre >= 0, pre, 0.2 * pre) - sh_ref[...]
    ws = jnp.exp(pre)
    tot = tot + jnp.dot(ws, u48_ref[...], preferred_element_type=f32) * h2e
    num = jnp.dot(tot, seln_ref[...], preferred_element_type=f32)
    den = jnp.dot(tot, seld_ref[...], preferred_element_type=f32)
    out_ref[...] = num / (den + 1e-16) + b2_ref[...]


def _sc_edge_kernel(widths, n_nodes, n_edges,
                    rows_hbm, adt_hbm, sh_hbm, src_hbm, dst_hbm, nd_hbm,
                    src0, src1, dst0, dst1, rows0, rows1, ad0, ad1,
                    sh_v, w_v, acc, semg0, semg1, sems0, sems1):
    row_w, a_cols, as_col = widths
    srcs, dsts, rows, ads = (src0, src1), (dst0, dst1), (rows0, rows1), (ad0, ad1)
    semg, sems = (semg0, semg1), (sems0, sems1)
    c = lax.axis_index("c")
    s = lax.axis_index("s")
    n_per_tile = _pad_rows(n_nodes) // NSUB
    r0 = s * n_per_tile
    e_per_w = n_edges // (NCORES * NSUB)
    chunks = e_per_w // EDGE_BLK
    e0 = (c * NSUB + s) * e_per_w
    iota = lax.broadcasted_iota(i32, (LANES,), 0)

    pltpu.sync_copy(sh_hbm, sh_v)

    zv = jnp.zeros((LANES,), f32)

    def zrow(j, carry):
        for off in range(0, row_w - LANES + 1, LANES):
            rows0[j, pl.ds(off, LANES)] = zv
        if row_w % LANES:
            rows0[j, pl.ds(row_w - LANES, LANES)] = zv
        return carry

    lax.fori_loop(0, EDGE_BLK, zrow, None)
    for off in range(0, n_per_tile, EDGE_BLK):
        sz = min(EDGE_BLK, n_per_tile - off)
        pltpu.sync_copy(rows0.at[pl.ds(0, sz)], acc.at[pl.ds(r0 + off, sz)])
    plsc.subcore_barrier()

    shv = sh_v[...]

    def load_idx(b, k):
        base = e0 + k * EDGE_BLK
        pltpu.sync_copy(src_hbm.at[pl.ds(base, EDGE_BLK)], srcs[b])
        pltpu.sync_copy(dst_hbm.at[pl.ds(base, EDGE_BLK)], dsts[b])

    def start_gather(b):
        return (pltpu.async_copy(rows_hbm.at[srcs[b]], rows[b], semg[b]),
                pltpu.async_copy(adt_hbm.at[dsts[b]], ads[b], semg[b]))

    def compute(b):
        rows_v = rows[b]
        adrows_v = ads[b]

        def wpass(i):
            p = iota + i * LANES
            if a_cols > 1:
                j = p >> 3
                hd = p & (a_cols - 1)
            else:
                j = p
                hd = jnp.zeros((LANES,), i32)
            asv = plsc.load_gather(rows_v, [j, hd + as_col])
            adv = plsc.load_gather(adrows_v, [j, hd])
            ev = asv + adv
            ev = jnp.where(ev >= 0, ev, ev * 0.2) - shv
            w = jnp.exp(ev)
            if a_cols > 1:
                plsc.store_scatter(rows_v, [j, hd + as_col], w)
            else:
                w_v[pl.ds(i * LANES, LANES)] = w

        plsc.parallel_loop(0, EDGE_BLK * a_cols // LANES, unroll=2)(wpass)

        lane0 = as_col - (row_w - LANES)

        if a_cols > 1:
            def mpass(j):
                w16 = rows_v[j, pl.ds(row_w - LANES, LANES)]
                for cc in range((row_w - a_cols) // LANES):
                    pat = (iota >> 3) + (lane0 + 2 * cc)
                    wv = jnp.take_along_axis(w16, pat, axis=0,
                                             mode="promise_in_bounds")
                    hv = rows_v[j, pl.ds(cc * LANES, LANES)]
                    rows_v[j, pl.ds(cc * LANES, LANES)] = hv * wv

            plsc.parallel_loop(0, EDGE_BLK, unroll=4)(mpass)
        else:
            def mpass(i):
                w16 = w_v[pl.ds(i * LANES, LANES)]
                for l in range(LANES):
                    wb = jnp.full((LANES,), w16[l], f32)
                    j = i * LANES + l
                    for cc in range(row_w // LANES):
                        hv = rows_v[j, pl.ds(cc * LANES, LANES)]
                        rows_v[j, pl.ds(cc * LANES, LANES)] = hv * wb

            plsc.parallel_loop(0, EDGE_BLK // LANES)(mpass)

    def start_scatter(b):
        return pltpu.async_copy(rows[b], acc.at[dsts[b]], sems[b], add=True)

    load_idx(0, 0)
    g = {0: start_gather(0)}
    sc = {}
    for k in range(chunks):
        b = k & 1
        nb = 1 - b
        if k + 1 < chunks:
            if k >= 1:
                sc.pop(nb).wait()
            load_idx(nb, k + 1)
            g[nb] = start_gather(nb)
        for h in g.pop(b):
            h.wait()
        compute(b)
        sc[b] = start_scatter(b)
    for h in sc.values():
        h.wait()
    plsc.subcore_barrier()
    pltpu.sync_copy(acc.at[pl.ds(r0, n_per_tile)],
                    nd_hbm.at[c, pl.ds(r0, n_per_tile)])


def _pad_rows(n_nodes):
    per_tile = -(-n_nodes // NSUB)
    per_tile = -(-per_tile // 8) * 8
    return NSUB * per_tile


def _make_sc_edge(n_nodes, n_edges, row_w, a_cols, as_col):
    mesh = plsc.VectorSubcoreMesh(core_axis_name="c", subcore_axis_name="s")
    n_pad = _pad_rows(n_nodes)
    return functools.partial(
        pl.kernel,
        functools.partial(_sc_edge_kernel, (row_w, a_cols, as_col),
                          n_nodes, n_edges),
        out_type=jax.ShapeDtypeStruct((NCORES, n_pad, row_w), f32),
        mesh=mesh,
        compiler_params=pltpu.CompilerParams(needs_layout_passes=False,
                                             use_tc_tiling_on_sc=False),
        scratch_types=[
            pltpu.VMEM((EDGE_BLK,), i32),
            pltpu.VMEM((EDGE_BLK,), i32),
            pltpu.VMEM((EDGE_BLK,), i32),
            pltpu.VMEM((EDGE_BLK,), i32),
            pltpu.VMEM((EDGE_BLK, row_w), f32),
            pltpu.VMEM((EDGE_BLK, row_w), f32),
            pltpu.VMEM((EDGE_BLK, 8), f32),
            pltpu.VMEM((EDGE_BLK, 8), f32),
            pltpu.VMEM((LANES,), f32),
            pltpu.VMEM((EDGE_BLK + LANES,), f32),
            pltpu.VMEM_SHARED((n_pad, row_w), f32),
            pltpu.SemaphoreType.DMA,
            pltpu.SemaphoreType.DMA,
            pltpu.SemaphoreType.DMA,
            pltpu.SemaphoreType.DMA,
        ],
    )()


def kernel(x, edge_index, W1, att_src1, att_dst1, b1,
           W2, att_src2, att_dst2, b2):
    N, D = x.shape
    E = edge_index.shape[1]
    H, HID = att_src1.shape
    F1 = H * HID
    C2 = W2.shape[1]
    R1W = F1 + H
    R2W = 48

    src = edge_index[0]
    dst = edge_index[1]

    af = att_src1.reshape(F1)
    adf = att_dst1.reshape(F1)
    rng = jnp.arange(F1)
    S = jnp.zeros((F1, H), f32).at[rng, rng // HID].set(1.0)
    W1ext = jnp.concatenate([W1, W1 @ (S * af[:, None])], axis=1)
    W1ad = W1 @ (S * adf[:, None])
    W1asad = jnp.concatenate([W1 @ (S * af[:, None]),
                              W1 @ (S * adf[:, None])], axis=1)

    selsum = jnp.concatenate([jnp.eye(H, dtype=f32),
                              jnp.eye(H, dtype=f32)], axis=0)
    sel64 = jnp.concatenate([jnp.eye(F1, dtype=f32),
                             jnp.zeros((H, F1), f32)], axis=0)
    sel8 = jnp.concatenate([jnp.zeros((F1, H), f32),
                            jnp.eye(H, dtype=f32)], axis=0)
    rep = S.T

    E2 = jnp.zeros((C2, R2W), f32).at[:, :C2].set(jnp.eye(C2, dtype=f32))
    E2 = E2.at[:, C2 + 1].set(att_src2[0])
    W2e = W2 @ E2
    C2row = jnp.zeros((1, R2W), f32).at[0, C2].set(1.0)
    W2ad = jnp.concatenate([W2 @ att_dst2.T,
                            jnp.zeros((F1, 7), f32)], axis=1)
    W2asad = jnp.zeros((F1, 16), f32)
    W2asad = W2asad.at[:, 0].set((W2 @ att_src2.T)[:, 0])
    W2asad = W2asad.at[:, 8].set((W2 @ att_dst2.T)[:, 0])

    sel41 = jnp.zeros((R2W, 8), f32).at[C2 + 1, :].set(1.0)
    e00 = jnp.zeros((8, 8), f32).at[0, :].set(1.0)
    u48 = jnp.zeros((8, R2W), f32).at[0, :].set(1.0)
    seln = jnp.zeros((R2W, C2), f32).at[:C2, :].set(jnp.eye(C2, dtype=f32))
    seld = jnp.zeros((R2W, C2), f32).at[C2, :].set(1.0)

    b1row = b1.reshape(1, F1)
    b2row = b2.reshape(1, C2)

    R = 2000
    grid = (N // R,)
    full = lambda shape: pl.BlockSpec(shape, lambda i: (0,) * len(shape))
    h1ext, ad1, asad1, mx1 = pl.pallas_call(
        _tc_proj_kernel,
        grid=grid,
        in_specs=[pl.BlockSpec((R, D), lambda i: (i, 0)),
                  full((D, R1W)), full((D, H)), full((D, 2 * H))],
        out_specs=[pl.BlockSpec((R, R1W), lambda i: (i, 0)),
                   pl.BlockSpec((R, H), lambda i: (i, 0)),
                   pl.BlockSpec((R, 2 * H), lambda i: (i, 0)),
                   full((1, 2 * H))],
        out_shape=[jax.ShapeDtypeStruct((N, R1W), f32),
                   jax.ShapeDtypeStruct((N, H), f32),
                   jax.ShapeDtypeStruct((N, 2 * H), f32),
                   jax.ShapeDtypeStruct((1, 2 * H), f32)],
    )(x, W1ext, W1ad, W1asad)

    s8 = mx1[0, :H] + mx1[0, H:]
    s8 = jnp.where(s8 >= 0, s8, 0.2 * s8)
    shift16 = jnp.tile(s8, 2)
    shift8 = s8.reshape(1, H)

    nd1 = _make_sc_edge(N, E, R1W, H, F1)(h1ext, ad1, shift16, src, dst)

    h2ext, ad2, mx2 = pl.pallas_call(
        _tc_mid_kernel,
        grid=grid,
        in_specs=[pl.BlockSpec((NCORES, R, R1W), lambda i: (0, i, 0)),
                  pl.BlockSpec((R, R1W), lambda i: (i, 0)),
                  pl.BlockSpec((R, 2 * H), lambda i: (i, 0)),
                  full((1, H)), full((1, F1)),
                  full((2 * H, H)), full((R1W, F1)), full((R1W, H)),
                  full((H, F1)),
                  full((F1, R2W)), full((1, R2W)), full((F1, 8)),
                  full((F1, 16))],
        out_specs=[pl.BlockSpec((R, R2W), lambda i: (i, 0)),
                   pl.BlockSpec((R, 8), lambda i: (i, 0)),
                   full((1, 16))],
        out_shape=[jax.ShapeDtypeStruct((N, R2W), f32),
                   jax.ShapeDtypeStruct((N, 8), f32),
                   jax.ShapeDtypeStruct((1, 16), f32)],
    )(nd1, h1ext, asad1, shift8, b1row,
      selsum, sel64, sel8, rep, W2e, C2row, W2ad, W2asad)

    s2 = mx2[0, 0] + mx2[0, 8]
    s2 = jnp.where(s2 >= 0, s2, 0.2 * s2)
    shift2_16 = jnp.full((LANES,), s2, f32)
    shift2_8 = jnp.full((1, 8), s2, f32)

    nd2 = _make_sc_edge(N, E, R2W, 1, C2 + 1)(h2ext, ad2, shift2_16, src, dst)

    out = pl.pallas_call(
        _tc_out_kernel,
        grid=grid,
        in_specs=[pl.BlockSpec((NCORES, R, R2W), lambda i: (0, i, 0)),
                  pl.BlockSpec((R, R2W), lambda i: (i, 0)),
                  pl.BlockSpec((R, 8), lambda i: (i, 0)),
                  full((1, 8)), full((1, C2)),
                  full((R2W, 8)), full((8, 8)), full((8, R2W)),
                  full((R2W, C2)), full((R2W, C2))],
        out_specs=pl.BlockSpec((R, C2), lambda i: (i, 0)),
        out_shape=jax.ShapeDtypeStruct((N, C2), f32),
    )(nd2, h2ext, ad2, shift2_8, b2row,
      sel41, e00, u48, seln, seld)
    return out


# speedup vs baseline: 1.0923x; 1.0923x over previous
import functools

import jax
import jax.numpy as jnp
from jax import lax
from jax.experimental import pallas as pl
from jax.experimental.pallas import tpu as pltpu
from jax.experimental.pallas import tpu_sc as plsc

f32 = jnp.float32
i32 = jnp.int32

NCORES = 2
NSUB = 16
LANES = 16

EDGE_BLK = 400
GROUP = 5


def _tc_proj_kernel(x_ref, wext_ref, wad_ref, wasad_ref,
                    h1ext_ref, ad1_ref, asad_ref, mx_ref):
    i = pl.program_id(0)
    xb = x_ref[...]
    h1ext_ref[...] = jnp.dot(xb, wext_ref[...], preferred_element_type=f32)
    ad1_ref[...] = jnp.dot(xb, wad_ref[...], preferred_element_type=f32)
    asad = jnp.dot(xb, wasad_ref[...], preferred_element_type=f32)
    asad_ref[...] = asad
    m = jnp.max(asad, axis=0, keepdims=True)
    prev = jnp.where(i == 0, jnp.full_like(m, -jnp.inf), mx_ref[...])
    mx_ref[...] = jnp.maximum(prev, m)


def _tc_mid_kernel(nd_ref, h1ext_ref, asad_ref, sh_ref, b1_ref,
                   selsum_ref, sel64_ref, sel8_ref, rep_ref,
                   w2e_ref, c2_ref, w2ad_ref, w2asad_ref,
                   h2ext_ref, ad2_ref, mx_ref):
    i = pl.program_id(0)
    nd = nd_ref[...]
    tot = nd[0] + nd[1]
    h1e = h1ext_ref[...]
    asad = asad_ref[...]
    pre = jnp.dot(asad, selsum_ref[...], preferred_element_type=f32)
    pre = jnp.where(pre >= 0, pre, 0.2 * pre) - sh_ref[...]
    ws = jnp.exp(pre)
    h64 = jnp.dot(h1e, sel64_ref[...], preferred_element_type=f32)
    ws64 = jnp.dot(ws, rep_ref[...], preferred_element_type=f32)
    num = jnp.dot(tot, sel64_ref[...], preferred_element_type=f32) + ws64 * h64
    den = jnp.dot(tot, sel8_ref[...], preferred_element_type=f32) + ws
    den64 = jnp.dot(den, rep_ref[...], preferred_element_type=f32)
    out1 = num / (den64 + 1e-16) + b1_ref[...]
    hg = jnp.where(out1 > 0, out1, jnp.exp(jnp.minimum(out1, 0.0)) - 1.0)
    h2e = jnp.dot(hg, w2e_ref[...], preferred_element_type=f32) + c2_ref[...]
    h2ext_ref[...] = h2e
    ad2_ref[...] = jnp.dot(hg, w2ad_ref[...], preferred_element_type=f32)
    asad2 = jnp.dot(hg, w2asad_ref[...], preferred_element_type=f32)
    m = jnp.max(asad2, axis=0, keepdims=True)
    prev = jnp.where(i == 0, jnp.full_like(m, -jnp.inf), mx_ref[...])
    mx_ref[...] = jnp.maximum(prev, m)


def _tc_out_kernel(nd_ref, h2ext_ref, ad2_ref, sh_ref, b2_ref,
                   sel41_ref, e00_ref, u48_ref, seln_ref, seld_ref,
                   out_ref):
    nd = nd_ref[...]
    tot = nd[0] + nd[1]
    h2e = h2ext_ref[...]
    as2 = jnp.dot(h2e, sel41_ref[...], preferred_element_type=f32)
    ad2 = jnp.dot(ad2_ref[...], e00_ref[...], preferred_element_type=f32)
    pre = as2 + ad2
    pre = jnp.where(pre >= 0, pre, 0.2 * pre) - sh_ref[...]
    ws = jnp.exp(pre)
    tot = tot + jnp.dot(ws, u48_ref[...], preferred_element_type=f32) * h2e
    num = jnp.dot(tot, seln_ref[...], preferred_element_type=f32)
    den = jnp.dot(tot, seld_ref[...], preferred_element_type=f32)
    out_ref[...] = num / (den + 1e-16) + b2_ref[...]


def _sc_edge_kernel(widths, n_nodes, n_edges,
                    rows_hbm, adt_hbm, sh_hbm, src_hbm, dst_hbm, nd_hbm,
                    src_g, dst_g, rows0, rows1, ad0, ad1,
                    sh_v, w_v, acc, semg0, semg1, sems0, sems1):
    row_w, a_cols, as_col = widths
    rows, ads = (rows0, rows1), (ad0, ad1)
    semg, sems = (semg0, semg1), (sems0, sems1)
    c = lax.axis_index("c")
    s = lax.axis_index("s")
    n_per_tile = _pad_rows(n_nodes) // NSUB
    r0 = s * n_per_tile
    e_per_w = n_edges // (NCORES * NSUB)
    chunks = e_per_w // EDGE_BLK
    iota = lax.broadcasted_iota(i32, (LANES,), 0)

    pltpu.sync_copy(sh_hbm, sh_v)

    zv = jnp.zeros((LANES,), f32)

    def zrow(j, carry):
        for off in range(0, row_w - LANES + 1, LANES):
            rows0[j, pl.ds(off, LANES)] = zv
        if row_w % LANES:
            rows0[j, pl.ds(row_w - LANES, LANES)] = zv
        return carry

    lax.fori_loop(0, EDGE_BLK, zrow, None)
    for off in range(0, n_per_tile, EDGE_BLK):
        sz = min(EDGE_BLK, n_per_tile - off)
        pltpu.sync_copy(rows0.at[pl.ds(0, sz)], acc.at[pl.ds(r0 + off, sz)])
    plsc.subcore_barrier()

    shv = sh_v[...]
    kc0 = (c * NSUB + s) * chunks

    def load_group(gb, grp):
        n = min(GROUP, chunks - grp * GROUP)
        pltpu.sync_copy(src_hbm.at[pl.ds(kc0 + grp * GROUP, n)],
                        src_g.at[gb, pl.ds(0, n)])
        pltpu.sync_copy(dst_hbm.at[pl.ds(kc0 + grp * GROUP, n)],
                        dst_g.at[gb, pl.ds(0, n)])

    def start_gather(b, k):
        gb, sl = (k // GROUP) & 1, k % GROUP
        return (pltpu.async_copy(rows_hbm.at[src_g.at[gb, sl]],
                                 rows[b], semg[b]),
                pltpu.async_copy(adt_hbm.at[dst_g.at[gb, sl]],
                                 ads[b], semg[b]))

    def compute(b):
        rows_v = rows[b]
        adrows_v = ads[b]

        def wpass(i):
            p = iota + i * LANES
            if a_cols > 1:
                j = p >> 3
                hd = p & (a_cols - 1)
            else:
                j = p
                hd = jnp.zeros((LANES,), i32)
            asv = plsc.load_gather(rows_v, [j, hd + as_col])
            adv = plsc.load_gather(adrows_v, [j, hd])
            ev = asv + adv
            ev = jnp.where(ev >= 0, ev, ev * 0.2) - shv
            w = jnp.exp(ev)
            if a_cols > 1:
                plsc.store_scatter(rows_v, [j, hd + as_col], w)
            else:
                w_v[pl.ds(i * LANES, LANES)] = w

        plsc.parallel_loop(0, EDGE_BLK * a_cols // LANES, unroll=2)(wpass)

        lane0 = as_col - (row_w - LANES)

        if a_cols > 1:
            def mpass(j):
                w16 = rows_v[j, pl.ds(row_w - LANES, LANES)]
                for cc in range((row_w - a_cols) // LANES):
                    pat = (iota >> 3) + (lane0 + 2 * cc)
                    wv = jnp.take_along_axis(w16, pat, axis=0,
                                             mode="promise_in_bounds")
                    hv = rows_v[j, pl.ds(cc * LANES, LANES)]
                    rows_v[j, pl.ds(cc * LANES, LANES)] = hv * wv

            plsc.parallel_loop(0, EDGE_BLK, unroll=2)(mpass)
        else:
            def mpass(i):
                w16 = w_v[pl.ds(i * LANES, LANES)]
                for l in range(LANES):
                    wb = jnp.full((LANES,), w16[l], f32)
                    j = i * LANES + l
                    for cc in range(row_w // LANES):
                        hv = rows_v[j, pl.ds(cc * LANES, LANES)]
                        rows_v[j, pl.ds(cc * LANES, LANES)] = hv * wb

            plsc.parallel_loop(0, EDGE_BLK // LANES)(mpass)

    def start_scatter(b, k):
        gb, sl = (k // GROUP) & 1, k % GROUP
        return pltpu.async_copy(rows[b], acc.at[dst_g.at[gb, sl]],
                                sems[b], add=True)

    load_group(0, 0)
    g = {0: start_gather(0, 0)}
    sc = {}
    for k in range(chunks):
        b = k & 1
        nb = 1 - b
        if k + 1 < chunks:
            if k >= 1:
                sc.pop(nb).wait()
            if (k + 1) % GROUP == 0:
                load_group(((k + 1) // GROUP) & 1, (k + 1) // GROUP)
            g[nb] = start_gather(nb, k + 1)
        for h in g.pop(b):
            h.wait()
        compute(b)
        sc[b] = start_scatter(b, k)
    for h in sc.values():
        h.wait()
    plsc.subcore_barrier()
    pltpu.sync_copy(acc.at[pl.ds(r0, n_per_tile)],
                    nd_hbm.at[c, pl.ds(r0, n_per_tile)])


def _pad_rows(n_nodes):
    per_tile = -(-n_nodes // NSUB)
    per_tile = -(-per_tile // 8) * 8
    return NSUB * per_tile


def _make_sc_edge(n_nodes, n_edges, row_w, a_cols, as_col):
    mesh = plsc.VectorSubcoreMesh(core_axis_name="c", subcore_axis_name="s")
    n_pad = _pad_rows(n_nodes)
    return functools.partial(
        pl.kernel,
        functools.partial(_sc_edge_kernel, (row_w, a_cols, as_col),
                          n_nodes, n_edges),
        out_type=jax.ShapeDtypeStruct((NCORES, n_pad, row_w), f32),
        mesh=mesh,
        compiler_params=pltpu.CompilerParams(needs_layout_passes=False,
                                             use_tc_tiling_on_sc=False),
        scratch_types=[
            pltpu.VMEM((2, GROUP, EDGE_BLK), i32),
            pltpu.VMEM((2, GROUP, EDGE_BLK), i32),
            pltpu.VMEM((EDGE_BLK, row_w), f32),
            pltpu.VMEM((EDGE_BLK, row_w), f32),
            pltpu.VMEM((EDGE_BLK, 8), f32),
            pltpu.VMEM((EDGE_BLK, 8), f32),
            pltpu.VMEM((LANES,), f32),
            pltpu.VMEM((EDGE_BLK + LANES,), f32),
            pltpu.VMEM_SHARED((n_pad, row_w), f32),
            pltpu.SemaphoreType.DMA,
            pltpu.SemaphoreType.DMA,
            pltpu.SemaphoreType.DMA,
            pltpu.SemaphoreType.DMA,
        ],
    )()


def kernel(x, edge_index, W1, att_src1, att_dst1, b1,
           W2, att_src2, att_dst2, b2):
    N, D = x.shape
    E = edge_index.shape[1]
    H, HID = att_src1.shape
    F1 = H * HID
    C2 = W2.shape[1]
    R1W = F1 + H
    R2W = 48

    src = edge_index[0].reshape(E // EDGE_BLK, EDGE_BLK)
    dst = edge_index[1].reshape(E // EDGE_BLK, EDGE_BLK)

    af = att_src1.reshape(F1)
    adf = att_dst1.reshape(F1)
    rng = jnp.arange(F1)
    S = jnp.zeros((F1, H), f32).at[rng, rng // HID].set(1.0)
    W1ext = jnp.concatenate([W1, W1 @ (S * af[:, None])], axis=1)
    W1ad = W1 @ (S * adf[:, None])
    W1asad = jnp.concatenate([W1 @ (S * af[:, None]),
                              W1 @ (S * adf[:, None])], axis=1)

    selsum = jnp.concatenate([jnp.eye(H, dtype=f32),
                              jnp.eye(H, dtype=f32)], axis=0)
    sel64 = jnp.concatenate([jnp.eye(F1, dtype=f32),
                             jnp.zeros((H, F1), f32)], axis=0)
    sel8 = jnp.concatenate([jnp.zeros((F1, H), f32),
                            jnp.eye(H, dtype=f32)], axis=0)
    rep = S.T

    E2 = jnp.zeros((C2, R2W), f32).at[:, :C2].set(jnp.eye(C2, dtype=f32))
    E2 = E2.at[:, C2 + 1].set(att_src2[0])
    W2e = W2 @ E2
    C2row = jnp.zeros((1, R2W), f32).at[0, C2].set(1.0)
    W2ad = jnp.concatenate([W2 @ att_dst2.T,
                            jnp.zeros((F1, 7), f32)], axis=1)
    W2asad = jnp.zeros((F1, 16), f32)
    W2asad = W2asad.at[:, 0].set((W2 @ att_src2.T)[:, 0])
    W2asad = W2asad.at[:, 8].set((W2 @ att_dst2.T)[:, 0])

    sel41 = jnp.zeros((R2W, 8), f32).at[C2 + 1, :].set(1.0)
    e00 = jnp.zeros((8, 8), f32).at[0, :].set(1.0)
    u48 = jnp.zeros((8, R2W), f32).at[0, :].set(1.0)
    seln = jnp.zeros((R2W, C2), f32).at[:C2, :].set(jnp.eye(C2, dtype=f32))
    seld = jnp.zeros((R2W, C2), f32).at[C2, :].set(1.0)

    b1row = b1.reshape(1, F1)
    b2row = b2.reshape(1, C2)

    R = 2000
    grid = (N // R,)
    full = lambda shape: pl.BlockSpec(shape, lambda i: (0,) * len(shape))
    h1ext, ad1, asad1, mx1 = pl.pallas_call(
        _tc_proj_kernel,
        grid=grid,
        in_specs=[pl.BlockSpec((R, D), lambda i: (i, 0)),
                  full((D, R1W)), full((D, H)), full((D, 2 * H))],
        out_specs=[pl.BlockSpec((R, R1W), lambda i: (i, 0)),
                   pl.BlockSpec((R, H), lambda i: (i, 0)),
                   pl.BlockSpec((R, 2 * H), lambda i: (i, 0)),
                   full((1, 2 * H))],
        out_shape=[jax.ShapeDtypeStruct((N, R1W), f32),
                   jax.ShapeDtypeStruct((N, H), f32),
                   jax.ShapeDtypeStruct((N, 2 * H), f32),
                   jax.ShapeDtypeStruct((1, 2 * H), f32)],
    )(x, W1ext, W1ad, W1asad)

    s8 = mx1[0, :H] + mx1[0, H:]
    s8 = jnp.where(s8 >= 0, s8, 0.2 * s8)
    shift16 = jnp.tile(s8, 2)
    shift8 = s8.reshape(1, H)

    nd1 = _make_sc_edge(N, E, R1W, H, F1)(h1ext, ad1, shift16, src, dst)

    h2ext, ad2, mx2 = pl.pallas_call(
        _tc_mid_kernel,
        grid=grid,
        in_specs=[pl.BlockSpec((NCORES, R, R1W), lambda i: (0, i, 0)),
                  pl.BlockSpec((R, R1W), lambda i: (i, 0)),
                  pl.BlockSpec((R, 2 * H), lambda i: (i, 0)),
                  full((1, H)), full((1, F1)),
                  full((2 * H, H)), full((R1W, F1)), full((R1W, H)),
                  full((H, F1)),
                  full((F1, R2W)), full((1, R2W)), full((F1, 8)),
                  full((F1, 16))],
        out_specs=[pl.BlockSpec((R, R2W), lambda i: (i, 0)),
                   pl.BlockSpec((R, 8), lambda i: (i, 0)),
                   full((1, 16))],
        out_shape=[jax.ShapeDtypeStruct((N, R2W), f32),
                   jax.ShapeDtypeStruct((N, 8), f32),
                   jax.ShapeDtypeStruct((1, 16), f32)],
    )(nd1, h1ext, asad1, shift8, b1row,
      selsum, sel64, sel8, rep, W2e, C2row, W2ad, W2asad)

    s2 = mx2[0, 0] + mx2[0, 8]
    s2 = jnp.where(s2 >= 0, s2, 0.2 * s2)
    shift2_16 = jnp.full((LANES,), s2, f32)
    shift2_8 = jnp.full((1, 8), s2, f32)

    nd2 = _make_sc_edge(N, E, R2W, 1, C2 + 1)(h2ext, ad2, shift2_16, src, dst)

    out = pl.pallas_call(
        _tc_out_kernel,
        grid=grid,
        in_specs=[pl.BlockSpec((NCORES, R, R2W), lambda i: (0, i, 0)),
                  pl.BlockSpec((R, R2W), lambda i: (i, 0)),
                  pl.BlockSpec((R, 8), lambda i: (i, 0)),
                  full((1, 8)), full((1, C2)),
                  full((R2W, 8)), full((8, 8)), full((8, R2W)),
                  full((R2W, C2)), full((R2W, C2))],
        out_specs=pl.BlockSpec((R, C2), lambda i: (i, 0)),
        out_shape=jax.ShapeDtypeStruct((N, C2), f32),
    )(nd2, h2ext, ad2, shift2_8, b2row,
      sel41, e00, u48, seln, seld)
    return out


# speedup vs baseline: 1.0983x; 1.0055x over previous
import functools

import jax
import jax.numpy as jnp
from jax import lax
from jax.experimental import pallas as pl
from jax.experimental.pallas import tpu as pltpu
from jax.experimental.pallas import tpu_sc as plsc

f32 = jnp.float32
i32 = jnp.int32

NCORES = 2
NSUB = 16
LANES = 16

EDGE_BLK = 400
GROUP = 5


def _tc_proj_kernel(x_ref, wext_ref, wad_ref, wasad_ref,
                    h1ext_ref, ad1_ref, asad_ref, mx_ref):
    i = pl.program_id(0)
    xb = x_ref[...]
    h1ext_ref[...] = jnp.dot(xb, wext_ref[...], preferred_element_type=f32)
    ad1_ref[...] = jnp.dot(xb, wad_ref[...], preferred_element_type=f32)
    asad = jnp.dot(xb, wasad_ref[...], preferred_element_type=f32)
    asad_ref[...] = asad
    m = jnp.max(asad, axis=0, keepdims=True)
    prev = jnp.where(i == 0, jnp.full_like(m, -jnp.inf), mx_ref[...])
    mx_ref[...] = jnp.maximum(prev, m)


def _tc_mid_kernel(nd_ref, h1ext_ref, asad_ref, sh_ref, b1_ref,
                   selsum_ref, sel64_ref, sel8_ref, rep_ref,
                   w2e_ref, c2_ref, w2ad_ref, w2asad_ref,
                   h2ext_ref, ad2_ref, mx_ref):
    i = pl.program_id(0)
    nd = nd_ref[...]
    tot = nd[0] + nd[1]
    h1e = h1ext_ref[...]
    asad = asad_ref[...]
    pre = jnp.dot(asad, selsum_ref[...], preferred_element_type=f32)
    pre = jnp.where(pre >= 0, pre, 0.2 * pre) - sh_ref[...]
    ws = jnp.exp(pre)
    h64 = jnp.dot(h1e, sel64_ref[...], preferred_element_type=f32)
    ws64 = jnp.dot(ws, rep_ref[...], preferred_element_type=f32)
    num = jnp.dot(tot, sel64_ref[...], preferred_element_type=f32) + ws64 * h64
    den = jnp.dot(tot, sel8_ref[...], preferred_element_type=f32) + ws
    den64 = jnp.dot(den, rep_ref[...], preferred_element_type=f32)
    out1 = num / (den64 + 1e-16) + b1_ref[...]
    hg = jnp.where(out1 > 0, out1, jnp.exp(jnp.minimum(out1, 0.0)) - 1.0)
    h2e = jnp.dot(hg, w2e_ref[...], preferred_element_type=f32) + c2_ref[...]
    h2ext_ref[...] = h2e
    ad2_ref[...] = jnp.dot(hg, w2ad_ref[...], preferred_element_type=f32)
    asad2 = jnp.dot(hg, w2asad_ref[...], preferred_element_type=f32)
    m = jnp.max(asad2, axis=0, keepdims=True)
    prev = jnp.where(i == 0, jnp.full_like(m, -jnp.inf), mx_ref[...])
    mx_ref[...] = jnp.maximum(prev, m)


def _tc_out_kernel(nd_ref, h2ext_ref, ad2_ref, sh_ref, b2_ref,
                   sel41_ref, e00_ref, u48_ref, seln_ref, seld_ref,
                   out_ref):
    nd = nd_ref[...]
    tot = nd[0] + nd[1]
    h2e = h2ext_ref[...]
    as2 = jnp.dot(h2e, sel41_ref[...], preferred_element_type=f32)
    ad2 = jnp.dot(ad2_ref[...], e00_ref[...], preferred_element_type=f32)
    pre = as2 + ad2
    pre = jnp.where(pre >= 0, pre, 0.2 * pre) - sh_ref[...]
    ws = jnp.exp(pre)
    tot = tot + jnp.dot(ws, u48_ref[...], preferred_element_type=f32) * h2e
    num = jnp.dot(tot, seln_ref[...], preferred_element_type=f32)
    den = jnp.dot(tot, seld_ref[...], preferred_element_type=f32)
    out_ref[...] = num / (den + 1e-16) + b2_ref[...]


def _sc_edge_kernel(widths, n_nodes, n_edges,
                    rows_hbm, adt_hbm, sh_hbm, src_hbm, dst_hbm, nd_hbm,
                    src_g, dst_g, rows0, rows1, ad0, ad1,
                    sh_v, w_v, acc, semg0, semg1, sems0, sems1):
    row_w, a_cols, as_col = widths
    rows, ads = (rows0, rows1), (ad0, ad1)
    semg, sems = (semg0, semg1), (sems0, sems1)
    c = lax.axis_index("c")
    s = lax.axis_index("s")
    n_per_tile = _pad_rows(n_nodes) // NSUB
    r0 = s * n_per_tile
    e_per_w = n_edges // (NCORES * NSUB)
    chunks = e_per_w // EDGE_BLK
    iota = lax.broadcasted_iota(i32, (LANES,), 0)

    pltpu.sync_copy(sh_hbm, sh_v)

    zv = jnp.zeros((LANES,), f32)

    def zrow(j, carry):
        for off in range(0, row_w - LANES + 1, LANES):
            rows0[j, pl.ds(off, LANES)] = zv
        if row_w % LANES:
            rows0[j, pl.ds(row_w - LANES, LANES)] = zv
        return carry

    lax.fori_loop(0, EDGE_BLK, zrow, None)
    for off in range(0, n_per_tile, EDGE_BLK):
        sz = min(EDGE_BLK, n_per_tile - off)
        pltpu.sync_copy(rows0.at[pl.ds(0, sz)], acc.at[pl.ds(r0 + off, sz)])
    plsc.subcore_barrier()

    shv = sh_v[...]
    kc0 = (c * NSUB + s) * chunks

    def load_group(gb, grp):
        n = min(GROUP, chunks - grp * GROUP)
        pltpu.sync_copy(src_hbm.at[pl.ds(kc0 + grp * GROUP, n)],
                        src_g.at[gb, pl.ds(0, n)])
        pltpu.sync_copy(dst_hbm.at[pl.ds(kc0 + grp * GROUP, n)],
                        dst_g.at[gb, pl.ds(0, n)])

    def start_gather(b, k):
        gb, sl = (k // GROUP) & 1, k % GROUP
        return (pltpu.async_copy(rows_hbm.at[src_g.at[gb, sl]],
                                 rows[b], semg[b]),
                pltpu.async_copy(adt_hbm.at[dst_g.at[gb, sl]],
                                 ads[b], semg[b]))

    def compute(b):
        rows_v = rows[b]
        adrows_v = ads[b]

        def wpass(i):
            p = iota + i * LANES
            if a_cols > 1:
                j = p >> 3
                hd = p & (a_cols - 1)
            else:
                j = p
                hd = jnp.zeros((LANES,), i32)
            asv = plsc.load_gather(rows_v, [j, hd + as_col])
            adv = plsc.load_gather(adrows_v, [j, hd])
            ev = asv + adv
            ev = jnp.where(ev >= 0, ev, ev * 0.2) - shv
            w = jnp.exp(ev)
            if a_cols > 1:
                plsc.store_scatter(rows_v, [j, hd + as_col], w)
            else:
                w_v[pl.ds(i * LANES, LANES)] = w

        plsc.parallel_loop(0, EDGE_BLK * a_cols // LANES,
                           unroll=4 if a_cols > 1 else 2)(wpass)

        lane0 = as_col - (row_w - LANES)

        if a_cols > 1:
            def mpass(j):
                w16 = rows_v[j, pl.ds(row_w - LANES, LANES)]
                for cc in range((row_w - a_cols) // LANES):
                    pat = (iota >> 3) + (lane0 + 2 * cc)
                    wv = jnp.take_along_axis(w16, pat, axis=0,
                                             mode="promise_in_bounds")
                    hv = rows_v[j, pl.ds(cc * LANES, LANES)]
                    rows_v[j, pl.ds(cc * LANES, LANES)] = hv * wv

            plsc.parallel_loop(0, EDGE_BLK, unroll=4)(mpass)
        else:
            def mpass(i):
                w16 = w_v[pl.ds(i * LANES, LANES)]
                for l in range(LANES):
                    wb = jnp.full((LANES,), w16[l], f32)
                    j = i * LANES + l
                    for cc in range(row_w // LANES):
                        hv = rows_v[j, pl.ds(cc * LANES, LANES)]
                        rows_v[j, pl.ds(cc * LANES, LANES)] = hv * wb

            plsc.parallel_loop(0, EDGE_BLK // LANES)(mpass)

    def start_scatter(b, k):
        gb, sl = (k // GROUP) & 1, k % GROUP
        return pltpu.async_copy(rows[b], acc.at[dst_g.at[gb, sl]],
                                sems[b], add=True)

    load_group(0, 0)
    g = {0: start_gather(0, 0)}
    sc = {}
    for k in range(chunks):
        b = k & 1
        nb = 1 - b
        if k + 1 < chunks:
            if k >= 1:
                sc.pop(nb).wait()
            if (k + 1) % GROUP == 0:
                load_group(((k + 1) // GROUP) & 1, (k + 1) // GROUP)
            g[nb] = start_gather(nb, k + 1)
        for h in g.pop(b):
            h.wait()
        compute(b)
        sc[b] = start_scatter(b, k)
    for h in sc.values():
        h.wait()
    plsc.subcore_barrier()
    pltpu.sync_copy(acc.at[pl.ds(r0, n_per_tile)],
                    nd_hbm.at[c, pl.ds(r0, n_per_tile)])


def _pad_rows(n_nodes):
    per_tile = -(-n_nodes // NSUB)
    per_tile = -(-per_tile // 8) * 8
    return NSUB * per_tile


def _make_sc_edge(n_nodes, n_edges, row_w, a_cols, as_col):
    mesh = plsc.VectorSubcoreMesh(core_axis_name="c", subcore_axis_name="s")
    n_pad = _pad_rows(n_nodes)
    return functools.partial(
        pl.kernel,
        functools.partial(_sc_edge_kernel, (row_w, a_cols, as_col),
                          n_nodes, n_edges),
        out_type=jax.ShapeDtypeStruct((NCORES, n_pad, row_w), f32),
        mesh=mesh,
        compiler_params=pltpu.CompilerParams(needs_layout_passes=False,
                                             use_tc_tiling_on_sc=False),
        scratch_types=[
            pltpu.VMEM((2, GROUP, EDGE_BLK), i32),
            pltpu.VMEM((2, GROUP, EDGE_BLK), i32),
            pltpu.VMEM((EDGE_BLK, row_w), f32),
            pltpu.VMEM((EDGE_BLK, row_w), f32),
            pltpu.VMEM((EDGE_BLK, 8), f32),
            pltpu.VMEM((EDGE_BLK, 8), f32),
            pltpu.VMEM((LANES,), f32),
            pltpu.VMEM((EDGE_BLK + LANES,), f32),
            pltpu.VMEM_SHARED((n_pad, row_w), f32),
            pltpu.SemaphoreType.DMA,
            pltpu.SemaphoreType.DMA,
            pltpu.SemaphoreType.DMA,
            pltpu.SemaphoreType.DMA,
        ],
    )()


def kernel(x, edge_index, W1, att_src1, att_dst1, b1,
           W2, att_src2, att_dst2, b2):
    N, D = x.shape
    E = edge_index.shape[1]
    H, HID = att_src1.shape
    F1 = H * HID
    C2 = W2.shape[1]
    R1W = F1 + H
    R2W = 48

    src = edge_index[0].reshape(E // EDGE_BLK, EDGE_BLK)
    dst = edge_index[1].reshape(E // EDGE_BLK, EDGE_BLK)

    af = att_src1.reshape(F1)
    adf = att_dst1.reshape(F1)
    rng = jnp.arange(F1)
    S = jnp.zeros((F1, H), f32).at[rng, rng // HID].set(1.0)
    W1ext = jnp.concatenate([W1, W1 @ (S * af[:, None])], axis=1)
    W1ad = W1 @ (S * adf[:, None])
    W1asad = jnp.concatenate([W1 @ (S * af[:, None]),
                              W1 @ (S * adf[:, None])], axis=1)

    selsum = jnp.concatenate([jnp.eye(H, dtype=f32),
                              jnp.eye(H, dtype=f32)], axis=0)
    sel64 = jnp.concatenate([jnp.eye(F1, dtype=f32),
                             jnp.zeros((H, F1), f32)], axis=0)
    sel8 = jnp.concatenate([jnp.zeros((F1, H), f32),
                            jnp.eye(H, dtype=f32)], axis=0)
    rep = S.T

    E2 = jnp.zeros((C2, R2W), f32).at[:, :C2].set(jnp.eye(C2, dtype=f32))
    E2 = E2.at[:, C2 + 1].set(att_src2[0])
    W2e = W2 @ E2
    C2row = jnp.zeros((1, R2W), f32).at[0, C2].set(1.0)
    W2ad = jnp.concatenate([W2 @ att_dst2.T,
                            jnp.zeros((F1, 7), f32)], axis=1)
    W2asad = jnp.zeros((F1, 16), f32)
    W2asad = W2asad.at[:, 0].set((W2 @ att_src2.T)[:, 0])
    W2asad = W2asad.at[:, 8].set((W2 @ att_dst2.T)[:, 0])

    sel41 = jnp.zeros((R2W, 8), f32).at[C2 + 1, :].set(1.0)
    e00 = jnp.zeros((8, 8), f32).at[0, :].set(1.0)
    u48 = jnp.zeros((8, R2W), f32).at[0, :].set(1.0)
    seln = jnp.zeros((R2W, C2), f32).at[:C2, :].set(jnp.eye(C2, dtype=f32))
    seld = jnp.zeros((R2W, C2), f32).at[C2, :].set(1.0)

    b1row = b1.reshape(1, F1)
    b2row = b2.reshape(1, C2)

    R = 2000
    grid = (N // R,)
    full = lambda shape: pl.BlockSpec(shape, lambda i: (0,) * len(shape))
    h1ext, ad1, asad1, mx1 = pl.pallas_call(
        _tc_proj_kernel,
        grid=grid,
        in_specs=[pl.BlockSpec((R, D), lambda i: (i, 0)),
                  full((D, R1W)), full((D, H)), full((D, 2 * H))],
        out_specs=[pl.BlockSpec((R, R1W), lambda i: (i, 0)),
                   pl.BlockSpec((R, H), lambda i: (i, 0)),
                   pl.BlockSpec((R, 2 * H), lambda i: (i, 0)),
                   full((1, 2 * H))],
        out_shape=[jax.ShapeDtypeStruct((N, R1W), f32),
                   jax.ShapeDtypeStruct((N, H), f32),
                   jax.ShapeDtypeStruct((N, 2 * H), f32),
                   jax.ShapeDtypeStruct((1, 2 * H), f32)],
    )(x, W1ext, W1ad, W1asad)

    s8 = mx1[0, :H] + mx1[0, H:]
    s8 = jnp.where(s8 >= 0, s8, 0.2 * s8)
    shift16 = jnp.tile(s8, 2)
    shift8 = s8.reshape(1, H)

    nd1 = _make_sc_edge(N, E, R1W, H, F1)(h1ext, ad1, shift16, src, dst)

    h2ext, ad2, mx2 = pl.pallas_call(
        _tc_mid_kernel,
        grid=grid,
        in_specs=[pl.BlockSpec((NCORES, R, R1W), lambda i: (0, i, 0)),
                  pl.BlockSpec((R, R1W), lambda i: (i, 0)),
                  pl.BlockSpec((R, 2 * H), lambda i: (i, 0)),
                  full((1, H)), full((1, F1)),
                  full((2 * H, H)), full((R1W, F1)), full((R1W, H)),
                  full((H, F1)),
                  full((F1, R2W)), full((1, R2W)), full((F1, 8)),
                  full((F1, 16))],
        out_specs=[pl.BlockSpec((R, R2W), lambda i: (i, 0)),
                   pl.BlockSpec((R, 8), lambda i: (i, 0)),
                   full((1, 16))],
        out_shape=[jax.ShapeDtypeStruct((N, R2W), f32),
                   jax.ShapeDtypeStruct((N, 8), f32),
                   jax.ShapeDtypeStruct((1, 16), f32)],
    )(nd1, h1ext, asad1, shift8, b1row,
      selsum, sel64, sel8, rep, W2e, C2row, W2ad, W2asad)

    s2 = mx2[0, 0] + mx2[0, 8]
    s2 = jnp.where(s2 >= 0, s2, 0.2 * s2)
    shift2_16 = jnp.full((LANES,), s2, f32)
    shift2_8 = jnp.full((1, 8), s2, f32)

    nd2 = _make_sc_edge(N, E, R2W, 1, C2 + 1)(h2ext, ad2, shift2_16, src, dst)

    out = pl.pallas_call(
        _tc_out_kernel,
        grid=grid,
        in_specs=[pl.BlockSpec((NCORES, R, R2W), lambda i: (0, i, 0)),
                  pl.BlockSpec((R, R2W), lambda i: (i, 0)),
                  pl.BlockSpec((R, 8), lambda i: (i, 0)),
                  full((1, 8)), full((1, C2)),
                  full((R2W, 8)), full((8, 8)), full((8, R2W)),
                  full((R2W, C2)), full((R2W, C2))],
        out_specs=pl.BlockSpec((R, C2), lambda i: (i, 0)),
        out_shape=jax.ShapeDtypeStruct((N, C2), f32),
    )(nd2, h2ext, ad2, shift2_8, b2row,
      sel41, e00, u48, seln, seld)
    return out


# speedup vs baseline: 1.1190x; 1.0188x over previous
import functools

import jax
import jax.numpy as jnp
from jax import lax
from jax.experimental import pallas as pl
from jax.experimental.pallas import tpu as pltpu
from jax.experimental.pallas import tpu_sc as plsc

f32 = jnp.float32
i32 = jnp.int32

NCORES = 2
NSUB = 16
LANES = 16

EDGE_BLK = 400
GROUP = 5


def _tc_proj_kernel(x_ref, wext_ref, wad_ref, wasad_ref,
                    h1ext_ref, ad1_ref, asad_ref, mx_ref):
    i = pl.program_id(0)
    xb = x_ref[...]
    h1ext_ref[...] = jnp.dot(xb, wext_ref[...], preferred_element_type=f32)
    ad1_ref[...] = jnp.dot(xb, wad_ref[...], preferred_element_type=f32)
    asad = jnp.dot(xb, wasad_ref[...], preferred_element_type=f32)
    asad_ref[...] = asad
    m = jnp.max(asad, axis=0, keepdims=True)
    prev = jnp.where(i == 0, jnp.full_like(m, -jnp.inf), mx_ref[...])
    mx_ref[...] = jnp.maximum(prev, m)


def _tc_mid_kernel(nd_ref, h1ext_ref, asad_ref, sh_ref, b1_ref,
                   selsum_ref, sel64_ref, sel8_ref, rep_ref,
                   w2e_ref, c2_ref, w2ad_ref, w2asad_ref,
                   h2ext_ref, ad2_ref, mx_ref):
    i = pl.program_id(0)
    nd = nd_ref[...]
    tot = nd[0] + nd[1]
    h1e = h1ext_ref[...]
    asad = asad_ref[...]
    pre = jnp.dot(asad, selsum_ref[...], preferred_element_type=f32)
    pre = jnp.where(pre >= 0, pre, 0.2 * pre) - sh_ref[...]
    ws = jnp.exp(pre)
    h64 = jnp.dot(h1e, sel64_ref[...], preferred_element_type=f32)
    ws64 = jnp.dot(ws, rep_ref[...], preferred_element_type=f32)
    num = jnp.dot(tot, sel64_ref[...], preferred_element_type=f32) + ws64 * h64
    den = jnp.dot(tot, sel8_ref[...], preferred_element_type=f32) + ws
    den64 = jnp.dot(den, rep_ref[...], preferred_element_type=f32)
    out1 = num / (den64 + 1e-16) + b1_ref[...]
    hg = jnp.where(out1 > 0, out1, jnp.exp(jnp.minimum(out1, 0.0)) - 1.0)
    h2e = jnp.dot(hg, w2e_ref[...], preferred_element_type=f32) + c2_ref[...]
    h2ext_ref[...] = h2e
    ad2_ref[...] = jnp.dot(hg, w2ad_ref[...], preferred_element_type=f32)
    asad2 = jnp.dot(hg, w2asad_ref[...], preferred_element_type=f32)
    m = jnp.max(asad2, axis=0, keepdims=True)
    prev = jnp.where(i == 0, jnp.full_like(m, -jnp.inf), mx_ref[...])
    mx_ref[...] = jnp.maximum(prev, m)


def _tc_out_kernel(nd_ref, h2ext_ref, ad2_ref, sh_ref, b2_ref,
                   sel41_ref, e00_ref, u48_ref, seln_ref, seld_ref,
                   out_ref):
    nd = nd_ref[...]
    tot = nd[0] + nd[1]
    h2e = h2ext_ref[...]
    as2 = jnp.dot(h2e, sel41_ref[...], preferred_element_type=f32)
    ad2 = jnp.dot(ad2_ref[...], e00_ref[...], preferred_element_type=f32)
    pre = as2 + ad2
    pre = jnp.where(pre >= 0, pre, 0.2 * pre) - sh_ref[...]
    ws = jnp.exp(pre)
    tot = tot + jnp.dot(ws, u48_ref[...], preferred_element_type=f32) * h2e
    num = jnp.dot(tot, seln_ref[...], preferred_element_type=f32)
    den = jnp.dot(tot, seld_ref[...], preferred_element_type=f32)
    out_ref[...] = num / (den + 1e-16) + b2_ref[...]


def _sc_edge_kernel(widths, n_nodes, n_edges, *refs):
    row_w, a_cols, as_col, nbuf = widths
    (rows_hbm, adt_hbm, sh_hbm, src_hbm, dst_hbm, nd_hbm,
     src_g, dst_g) = refs[:8]
    rows = refs[8:8 + nbuf]
    ads = refs[8 + nbuf:8 + 2 * nbuf]
    sh_v, w_v, acc = refs[8 + 2 * nbuf:11 + 2 * nbuf]
    semg = refs[11 + 2 * nbuf:11 + 3 * nbuf]
    sems = refs[11 + 3 * nbuf:11 + 4 * nbuf]
    rows0 = rows[0]
    c = lax.axis_index("c")
    s = lax.axis_index("s")
    n_per_tile = _pad_rows(n_nodes) // NSUB
    r0 = s * n_per_tile
    e_per_w = n_edges // (NCORES * NSUB)
    chunks = e_per_w // EDGE_BLK
    iota = lax.broadcasted_iota(i32, (LANES,), 0)

    pltpu.sync_copy(sh_hbm, sh_v)

    zv = jnp.zeros((LANES,), f32)

    def zrow(j, carry):
        for off in range(0, row_w - LANES + 1, LANES):
            rows0[j, pl.ds(off, LANES)] = zv
        if row_w % LANES:
            rows0[j, pl.ds(row_w - LANES, LANES)] = zv
        return carry

    lax.fori_loop(0, EDGE_BLK, zrow, None)
    for off in range(0, n_per_tile, EDGE_BLK):
        sz = min(EDGE_BLK, n_per_tile - off)
        pltpu.sync_copy(rows0.at[pl.ds(0, sz)], acc.at[pl.ds(r0 + off, sz)])
    plsc.subcore_barrier()

    shv = sh_v[...]
    kc0 = (c * NSUB + s) * chunks

    def load_group(gb, grp):
        n = min(GROUP, chunks - grp * GROUP)
        pltpu.sync_copy(src_hbm.at[pl.ds(kc0 + grp * GROUP, n)],
                        src_g.at[gb, pl.ds(0, n)])
        pltpu.sync_copy(dst_hbm.at[pl.ds(kc0 + grp * GROUP, n)],
                        dst_g.at[gb, pl.ds(0, n)])

    def start_gather(b, k):
        gb, sl = (k // GROUP) & 1, k % GROUP
        return (pltpu.async_copy(rows_hbm.at[src_g.at[gb, sl]],
                                 rows[b], semg[b]),
                pltpu.async_copy(adt_hbm.at[dst_g.at[gb, sl]],
                                 ads[b], semg[b]))

    def compute(b):
        rows_v = rows[b]
        adrows_v = ads[b]

        def wpass(i):
            p = iota + i * LANES
            if a_cols > 1:
                j = p >> 3
                hd = p & (a_cols - 1)
            else:
                j = p
                hd = jnp.zeros((LANES,), i32)
            asv = plsc.load_gather(rows_v, [j, hd + as_col])
            adv = plsc.load_gather(adrows_v, [j, hd])
            ev = asv + adv
            ev = jnp.where(ev >= 0, ev, ev * 0.2) - shv
            w = jnp.exp(ev)
            if a_cols > 1:
                plsc.store_scatter(rows_v, [j, hd + as_col], w)
            else:
                w_v[pl.ds(i * LANES, LANES)] = w

        plsc.parallel_loop(0, EDGE_BLK * a_cols // LANES,
                           unroll=4 if a_cols > 1 else 2)(wpass)

        lane0 = as_col - (row_w - LANES)

        if a_cols > 1:
            def mpass(j):
                w16 = rows_v[j, pl.ds(row_w - LANES, LANES)]
                for cc in range((row_w - a_cols) // LANES):
                    pat = (iota >> 3) + (lane0 + 2 * cc)
                    wv = jnp.take_along_axis(w16, pat, axis=0,
                                             mode="promise_in_bounds")
                    hv = rows_v[j, pl.ds(cc * LANES, LANES)]
                    rows_v[j, pl.ds(cc * LANES, LANES)] = hv * wv

            plsc.parallel_loop(0, EDGE_BLK, unroll=4)(mpass)
        else:
            def mpass(i):
                w16 = w_v[pl.ds(i * LANES, LANES)]
                for l in range(LANES):
                    wb = jnp.full((LANES,), w16[l], f32)
                    j = i * LANES + l
                    for cc in range(row_w // LANES):
                        hv = rows_v[j, pl.ds(cc * LANES, LANES)]
                        rows_v[j, pl.ds(cc * LANES, LANES)] = hv * wb

            plsc.parallel_loop(0, EDGE_BLK // LANES)(mpass)

    def start_scatter(b, k):
        gb, sl = (k // GROUP) & 1, k % GROUP
        return pltpu.async_copy(rows[b], acc.at[dst_g.at[gb, sl]],
                                sems[b], add=True)

    load_group(0, 0)
    g = {0: start_gather(0, 0)}
    sc = {}
    for k in range(chunks):
        b = k % nbuf
        if k + 1 < chunks:
            nb = (k + 1) % nbuf
            if k + 1 >= nbuf:
                sc.pop(nb).wait()
            if (k + 1) % GROUP == 0:
                load_group(((k + 1) // GROUP) & 1, (k + 1) // GROUP)
            g[nb] = start_gather(nb, k + 1)
        for h in g.pop(b):
            h.wait()
        compute(b)
        sc[b] = start_scatter(b, k)
    for h in sc.values():
        h.wait()
    plsc.subcore_barrier()
    pltpu.sync_copy(acc.at[pl.ds(r0, n_per_tile)],
                    nd_hbm.at[c, pl.ds(r0, n_per_tile)])


def _pad_rows(n_nodes):
    per_tile = -(-n_nodes // NSUB)
    per_tile = -(-per_tile // 8) * 8
    return NSUB * per_tile


def _make_sc_edge(n_nodes, n_edges, row_w, a_cols, as_col, nbuf):
    mesh = plsc.VectorSubcoreMesh(core_axis_name="c", subcore_axis_name="s")
    n_pad = _pad_rows(n_nodes)
    return functools.partial(
        pl.kernel,
        functools.partial(_sc_edge_kernel, (row_w, a_cols, as_col, nbuf),
                          n_nodes, n_edges),
        out_type=jax.ShapeDtypeStruct((NCORES, n_pad, row_w), f32),
        mesh=mesh,
        compiler_params=pltpu.CompilerParams(needs_layout_passes=False,
                                             use_tc_tiling_on_sc=False),
        scratch_types=(
            [pltpu.VMEM((2, GROUP, EDGE_BLK), i32),
             pltpu.VMEM((2, GROUP, EDGE_BLK), i32)]
            + [pltpu.VMEM((EDGE_BLK, row_w), f32)] * nbuf
            + [pltpu.VMEM((EDGE_BLK, 8), f32)] * nbuf
            + [pltpu.VMEM((LANES,), f32),
               pltpu.VMEM((EDGE_BLK + LANES,), f32),
               pltpu.VMEM_SHARED((n_pad, row_w), f32)]
            + [pltpu.SemaphoreType.DMA] * (2 * nbuf)
        ),
    )()


def kernel(x, edge_index, W1, att_src1, att_dst1, b1,
           W2, att_src2, att_dst2, b2):
    N, D = x.shape
    E = edge_index.shape[1]
    H, HID = att_src1.shape
    F1 = H * HID
    C2 = W2.shape[1]
    R1W = F1 + H
    R2W = 48

    src = edge_index[0].reshape(E // EDGE_BLK, EDGE_BLK)
    dst = edge_index[1].reshape(E // EDGE_BLK, EDGE_BLK)

    af = att_src1.reshape(F1)
    adf = att_dst1.reshape(F1)
    rng = jnp.arange(F1)
    S = jnp.zeros((F1, H), f32).at[rng, rng // HID].set(1.0)
    W1ext = jnp.concatenate([W1, W1 @ (S * af[:, None])], axis=1)
    W1ad = W1 @ (S * adf[:, None])
    W1asad = jnp.concatenate([W1 @ (S * af[:, None]),
                              W1 @ (S * adf[:, None])], axis=1)

    selsum = jnp.concatenate([jnp.eye(H, dtype=f32),
                              jnp.eye(H, dtype=f32)], axis=0)
    sel64 = jnp.concatenate([jnp.eye(F1, dtype=f32),
                             jnp.zeros((H, F1), f32)], axis=0)
    sel8 = jnp.concatenate([jnp.zeros((F1, H), f32),
                            jnp.eye(H, dtype=f32)], axis=0)
    rep = S.T

    E2 = jnp.zeros((C2, R2W), f32).at[:, :C2].set(jnp.eye(C2, dtype=f32))
    E2 = E2.at[:, C2 + 1].set(att_src2[0])
    W2e = W2 @ E2
    C2row = jnp.zeros((1, R2W), f32).at[0, C2].set(1.0)
    W2ad = jnp.concatenate([W2 @ att_dst2.T,
                            jnp.zeros((F1, 7), f32)], axis=1)
    W2asad = jnp.zeros((F1, 16), f32)
    W2asad = W2asad.at[:, 0].set((W2 @ att_src2.T)[:, 0])
    W2asad = W2asad.at[:, 8].set((W2 @ att_dst2.T)[:, 0])

    sel41 = jnp.zeros((R2W, 8), f32).at[C2 + 1, :].set(1.0)
    e00 = jnp.zeros((8, 8), f32).at[0, :].set(1.0)
    u48 = jnp.zeros((8, R2W), f32).at[0, :].set(1.0)
    seln = jnp.zeros((R2W, C2), f32).at[:C2, :].set(jnp.eye(C2, dtype=f32))
    seld = jnp.zeros((R2W, C2), f32).at[C2, :].set(1.0)

    b1row = b1.reshape(1, F1)
    b2row = b2.reshape(1, C2)

    R = 2000
    grid = (N // R,)
    full = lambda shape: pl.BlockSpec(shape, lambda i: (0,) * len(shape))
    h1ext, ad1, asad1, mx1 = pl.pallas_call(
        _tc_proj_kernel,
        grid=grid,
        in_specs=[pl.BlockSpec((R, D), lambda i: (i, 0)),
                  full((D, R1W)), full((D, H)), full((D, 2 * H))],
        out_specs=[pl.BlockSpec((R, R1W), lambda i: (i, 0)),
                   pl.BlockSpec((R, H), lambda i: (i, 0)),
                   pl.BlockSpec((R, 2 * H), lambda i: (i, 0)),
                   full((1, 2 * H))],
        out_shape=[jax.ShapeDtypeStruct((N, R1W), f32),
                   jax.ShapeDtypeStruct((N, H), f32),
                   jax.ShapeDtypeStruct((N, 2 * H), f32),
                   jax.ShapeDtypeStruct((1, 2 * H), f32)],
    )(x, W1ext, W1ad, W1asad)

    s8 = mx1[0, :H] + mx1[0, H:]
    s8 = jnp.where(s8 >= 0, s8, 0.2 * s8)
    shift16 = jnp.tile(s8, 2)
    shift8 = s8.reshape(1, H)

    nd1 = _make_sc_edge(N, E, R1W, H, F1, 2)(h1ext, ad1, shift16, src, dst)

    h2ext, ad2, mx2 = pl.pallas_call(
        _tc_mid_kernel,
        grid=grid,
        in_specs=[pl.BlockSpec((NCORES, R, R1W), lambda i: (0, i, 0)),
                  pl.BlockSpec((R, R1W), lambda i: (i, 0)),
                  pl.BlockSpec((R, 2 * H), lambda i: (i, 0)),
                  full((1, H)), full((1, F1)),
                  full((2 * H, H)), full((R1W, F1)), full((R1W, H)),
                  full((H, F1)),
                  full((F1, R2W)), full((1, R2W)), full((F1, 8)),
                  full((F1, 16))],
        out_specs=[pl.BlockSpec((R, R2W), lambda i: (i, 0)),
                   pl.BlockSpec((R, 8), lambda i: (i, 0)),
                   full((1, 16))],
        out_shape=[jax.ShapeDtypeStruct((N, R2W), f32),
                   jax.ShapeDtypeStruct((N, 8), f32),
                   jax.ShapeDtypeStruct((1, 16), f32)],
    )(nd1, h1ext, asad1, shift8, b1row,
      selsum, sel64, sel8, rep, W2e, C2row, W2ad, W2asad)

    s2 = mx2[0, 0] + mx2[0, 8]
    s2 = jnp.where(s2 >= 0, s2, 0.2 * s2)
    shift2_16 = jnp.full((LANES,), s2, f32)
    shift2_8 = jnp.full((1, 8), s2, f32)

    nd2 = _make_sc_edge(N, E, R2W, 1, C2 + 1, 3)(h2ext, ad2, shift2_16,
                                                 src, dst)

    out = pl.pallas_call(
        _tc_out_kernel,
        grid=grid,
        in_specs=[pl.BlockSpec((NCORES, R, R2W), lambda i: (0, i, 0)),
                  pl.BlockSpec((R, R2W), lambda i: (i, 0)),
                  pl.BlockSpec((R, 8), lambda i: (i, 0)),
                  full((1, 8)), full((1, C2)),
                  full((R2W, 8)), full((8, 8)), full((8, R2W)),
                  full((R2W, C2)), full((R2W, C2))],
        out_specs=pl.BlockSpec((R, C2), lambda i: (i, 0)),
        out_shape=jax.ShapeDtypeStruct((N, C2), f32),
    )(nd2, h2ext, ad2, shift2_8, b2row,
      sel41, e00, u48, seln, seld)
    return out
